# baseline (speedup 1.0000x reference)
import functools

import jax
import jax.numpy as jnp
from jax import lax
from jax.experimental import pallas as pl
from jax.experimental.pallas import tpu as pltpu
from jax.experimental.pallas import tpu_sc as plsc

EMBED = 128
HALF = EMBED // 2
NH, NM = 24, 12
NHP, NMP = 32, 16
NCOMBO = NH * NM

CH = 128
NBP = 3


def _table_body(ht_ref, mt_ref, w_ref, b_ref, out_ref):
    w = w_ref[...]
    hp = lax.dot_general(ht_ref[...], w[:, :HALF], (((1,), (1,)), ((), ())),
                         preferred_element_type=jnp.float32,
                         precision=lax.Precision.HIGHEST)
    mp = lax.dot_general(mt_ref[...], w[:, HALF:], (((1,), (1,)), ((), ())),
                         preferred_element_type=jnp.float32,
                         precision=lax.Precision.HIGHEST)
    rh = lax.broadcasted_iota(jnp.int32, (NCOMBO, NHP), 0)
    ch = lax.broadcasted_iota(jnp.int32, (NCOMBO, NHP), 1)
    oh_h = jnp.where(rh // NM == ch, 1.0, 0.0).astype(jnp.float32)
    rm = lax.broadcasted_iota(jnp.int32, (NCOMBO, NMP), 0)
    cm = lax.broadcasted_iota(jnp.int32, (NCOMBO, NMP), 1)
    oh_m = jnp.where(rm % NM == cm, 1.0, 0.0).astype(jnp.float32)
    out = lax.dot_general(oh_h, hp, (((1,), (0,)), ((), ())),
                          preferred_element_type=jnp.float32,
                          precision=lax.Precision.HIGHEST)
    out = out + lax.dot_general(oh_m, mp, (((1,), (0,)), ((), ())),
                                preferred_element_type=jnp.float32,
                                precision=lax.Precision.HIGHEST)
    out_ref[...] = out + b_ref[...]


def _build_fused_table(hour_table, month_table, W, b):
    ht = jnp.zeros((NHP, HALF), jnp.float32).at[:NH].set(hour_table)
    mt = jnp.zeros((NMP, HALF), jnp.float32).at[:NM].set(month_table)
    return pl.pallas_call(
        _table_body,
        out_shape=jax.ShapeDtypeStruct((NCOMBO, EMBED), jnp.float32),
    )(ht, mt, W, b.reshape(1, EMBED))


def _idx_body(h_ref, m_ref, o_ref):
    o_ref[...] = h_ref[...] * NM + m_ref[...]


def _build_fused_idx(hour, month):
    return pl.pallas_call(
        _idx_body,
        out_shape=jax.ShapeDtypeStruct(hour.shape, jnp.int32),
    )(hour, month)


def _make_gather_kernel(n_tokens):
    info = plsc.get_sparse_core_info()
    nc, ns = info.num_cores, info.num_subcores
    nw = nc * ns
    npw = n_tokens // nw
    assert n_tokens % nw == 0 and npw % CH == 0
    nch = npw // CH
    npair = nch // 2
    assert nch % 2 == 0 and npair > 2 * NBP

    mesh = plsc.VectorSubcoreMesh(core_axis_name="c", subcore_axis_name="s")

    @functools.partial(
        pl.kernel,
        mesh=mesh,
        out_type=jax.ShapeDtypeStruct((n_tokens, EMBED), jnp.float32),
        scratch_types=[
            pltpu.VMEM((nch, CH), jnp.int32),
            pltpu.VMEM((NBP, 2 * CH, EMBED), jnp.float32),
            pltpu.VMEM_SHARED((NCOMBO, EMBED), jnp.float32),
            pltpu.SemaphoreType.DMA,
            pltpu.SemaphoreType.DMA,
        ],
    )
    def gather_kernel(fused_hbm, idx_hbm, out_hbm, idx_v, rows_v, tbl_sh,
                      sem_g, sem_s):
        wid = lax.axis_index("s") * nc + lax.axis_index("c")
        base = wid * npw
        @pl.when(lax.axis_index("s") == 0)
        def _():
            pltpu.sync_copy(fused_hbm, tbl_sh)
        pltpu.sync_copy(idx_hbm.at[pl.ds(wid * nch, nch)], idx_v)
        plsc.subcore_barrier()

        def start_gathers(p, slot_buf):
            pltpu.async_copy(tbl_sh.at[idx_v.at[2 * p]],
                             slot_buf.at[pl.ds(0, CH)], sem_g)
            pltpu.async_copy(tbl_sh.at[idx_v.at[2 * p + 1]],
                             slot_buf.at[pl.ds(CH, CH)], sem_g)

        def start_store(p, slot_buf):
            pltpu.async_copy(
                slot_buf, out_hbm.at[pl.ds(base + p * 2 * CH, 2 * CH)], sem_s)

        def wait_gathers(slot_buf):
            pltpu.make_async_copy(fused_hbm.at[pl.ds(0, CH)],
                                  slot_buf.at[pl.ds(0, CH)], sem_g).wait()
            pltpu.make_async_copy(fused_hbm.at[pl.ds(0, CH)],
                                  slot_buf.at[pl.ds(CH, CH)], sem_g).wait()

        def wait_store(slot_buf):
            pltpu.make_async_copy(slot_buf, out_hbm.at[pl.ds(0, 2 * CH)],
                                  sem_s).wait()

        start_gathers(0, rows_v.at[0])
        for p in range(2):
            start_gathers(p + 1, rows_v.at[p + 1])
            wait_gathers(rows_v.at[p])
            start_store(p, rows_v.at[p])

        def body(p, _):
            buf = rows_v.at[lax.rem(p, NBP)]
            nxt = rows_v.at[lax.rem(p + 1, NBP)]
            wait_store(nxt)
            start_gathers(p + 1, nxt)
            wait_gathers(buf)
            start_store(p, buf)
            return 0
        lax.fori_loop(2, npair - 1, body, 0)

        buf = rows_v.at[(npair - 1) % NBP]
        wait_gathers(buf)
        start_store(npair - 1, buf)
        for _ in range(3):
            wait_store(rows_v.at[0])

    return gather_kernel


def kernel(hour, month, hour_table, month_table, W, b):
    B, L = hour.shape
    n = B * L
    fused = _build_fused_table(hour_table, month_table, W, b)
    idx2d = _build_fused_idx(hour.astype(jnp.int32),
                             month.astype(jnp.int32)).reshape(n // CH, CH)
    gather = _make_gather_kernel(n)
    out = gather(fused, idx2d)
    return out.reshape(B, L, EMBED)


# speedup vs baseline: 1.0105x; 1.0105x over previous
import functools

import jax
import jax.numpy as jnp
from jax import lax
from jax.experimental import pallas as pl
from jax.experimental.pallas import tpu as pltpu
from jax.experimental.pallas import tpu_sc as plsc

EMBED = 128
HALF = EMBED // 2
NH, NM = 24, 12
NHP, NMP = 32, 16
NCOMBO = NH * NM

CH = 128
NB = 6
LOOKAHEAD = 3


def _prep_body(ht_ref, mt_ref, w_ref, b_ref, h_ref, m_ref, out_ref, idx_ref):
    idx_ref[...] = h_ref[...] * NM + m_ref[...]
    w = w_ref[...]
    hp = lax.dot_general(ht_ref[...], w[:, :HALF], (((1,), (1,)), ((), ())),
                         preferred_element_type=jnp.float32,
                         precision=lax.Precision.HIGHEST)
    mp = lax.dot_general(mt_ref[...], w[:, HALF:], (((1,), (1,)), ((), ())),
                         preferred_element_type=jnp.float32,
                         precision=lax.Precision.HIGHEST)
    rh = lax.broadcasted_iota(jnp.int32, (NCOMBO, NHP), 0)
    ch = lax.broadcasted_iota(jnp.int32, (NCOMBO, NHP), 1)
    oh_h = jnp.where(rh // NM == ch, 1.0, 0.0).astype(jnp.float32)
    rm = lax.broadcasted_iota(jnp.int32, (NCOMBO, NMP), 0)
    cm = lax.broadcasted_iota(jnp.int32, (NCOMBO, NMP), 1)
    oh_m = jnp.where(rm % NM == cm, 1.0, 0.0).astype(jnp.float32)
    out = lax.dot_general(oh_h, hp, (((1,), (0,)), ((), ())),
                          preferred_element_type=jnp.float32,
                          precision=lax.Precision.HIGHEST)
    out = out + lax.dot_general(oh_m, mp, (((1,), (0,)), ((), ())),
                                preferred_element_type=jnp.float32,
                                precision=lax.Precision.HIGHEST)
    out_ref[...] = out + b_ref[...]


def _build_prep(hour_table, month_table, W, b, hour, month):
    ht = jnp.zeros((NHP, HALF), jnp.float32).at[:NH].set(hour_table)
    mt = jnp.zeros((NMP, HALF), jnp.float32).at[:NM].set(month_table)
    return pl.pallas_call(
        _prep_body,
        out_shape=[jax.ShapeDtypeStruct((NCOMBO, EMBED), jnp.float32),
                   jax.ShapeDtypeStruct(hour.shape, jnp.int32)],
    )(ht, mt, W, b.reshape(1, EMBED), hour, month)


def _make_gather_kernel(n_tokens):
    info = plsc.get_sparse_core_info()
    nc, ns = info.num_cores, info.num_subcores
    nw = nc * ns
    npw = n_tokens // nw
    assert n_tokens % nw == 0 and npw % CH == 0
    nch = npw // CH
    assert nch > 2 * NB

    mesh = plsc.VectorSubcoreMesh(core_axis_name="c", subcore_axis_name="s")

    @functools.partial(
        pl.kernel,
        mesh=mesh,
        out_type=jax.ShapeDtypeStruct((n_tokens, EMBED), jnp.float32),
        scratch_types=[
            pltpu.VMEM((nch, CH), jnp.int32),
            pltpu.VMEM((NB, CH, EMBED), jnp.float32),
            pltpu.VMEM_SHARED((NCOMBO, EMBED), jnp.float32),
            pltpu.SemaphoreType.DMA,
            pltpu.SemaphoreType.DMA,
        ],
    )
    def gather_kernel(fused_hbm, idx_hbm, out_hbm, idx_v, rows_v, tbl_sh,
                      sem_g, sem_s):
        wid = lax.axis_index("s") * nc + lax.axis_index("c")
        base = wid * npw
        @pl.when(lax.axis_index("s") == 0)
        def _():
            pltpu.sync_copy(fused_hbm, tbl_sh)
        pltpu.sync_copy(idx_hbm.at[pl.ds(wid * nch, nch)], idx_v)
        plsc.subcore_barrier()

        def start_gather(g, slot_buf):
            pltpu.async_copy(tbl_sh.at[idx_v.at[g]], slot_buf, sem_g)

        def start_store(g, slot_buf):
            pltpu.async_copy(
                slot_buf, out_hbm.at[pl.ds(base + g * CH, CH)], sem_s)

        def wait_gather(slot_buf):
            pltpu.make_async_copy(fused_hbm.at[pl.ds(0, CH)], slot_buf,
                                  sem_g).wait()

        def wait_store(slot_buf):
            pltpu.make_async_copy(slot_buf, out_hbm.at[pl.ds(0, CH)],
                                  sem_s).wait()

        for g in range(LOOKAHEAD):
            start_gather(g, rows_v.at[g])
        for g in range(2):
            wait_gather(rows_v.at[g % NB])
            start_store(g, rows_v.at[g % NB])
            start_gather(g + LOOKAHEAD, rows_v.at[(g + LOOKAHEAD) % NB])

        def body(g, _):
            slot = lax.rem(g, NB)
            buf = rows_v.at[slot]
            nxt = rows_v.at[lax.rem(g + LOOKAHEAD, NB)]
            wait_gather(buf)
            start_store(g, buf)
            wait_store(buf)
            start_gather(g + LOOKAHEAD, nxt)
            return 0
        lax.fori_loop(2, nch - LOOKAHEAD, body, 0)

        for k in range(LOOKAHEAD):
            g = nch - LOOKAHEAD + k
            buf = rows_v.at[g % NB]
            wait_gather(buf)
            start_store(g, buf)
        for _ in range(2 + LOOKAHEAD):
            wait_store(rows_v.at[0])

    return gather_kernel


def kernel(hour, month, hour_table, month_table, W, b):
    B, L = hour.shape
    n = B * L
    fused, idxn = _build_prep(hour_table, month_table, W, b,
                              hour.astype(jnp.int32), month.astype(jnp.int32))
    idx2d = idxn.reshape(n // CH, CH)
    gather = _make_gather_kernel(n)
    out = gather(fused, idx2d)
    return out.reshape(B, L, EMBED)
